# baseline (speedup 1.0000x reference)
import functools

import jax
import jax.numpy as jnp
from jax import lax
from jax.experimental import pallas as pl
from jax.experimental.pallas import tpu as pltpu
from jax.experimental.pallas import tpu_sc as plsc

N = 10000
NP = 10240
D = 128
E = 320000
NC = 2
NS = 16
CHUNK = 128
IB = 14
NBLK = 6
CHUNKS = NBLK * IB
E_PAD = NC * NS * CHUNKS * CHUNK
RPT = NP // NS

_mesh = plsc.VectorSubcoreMesh(core_axis_name="c", subcore_axis_name="s")


def _sc_deg_body(ed_hbm, ones_hbm, zeros_hbm, out_hbm, dst_v, ones_v, deg_sh):
    c = lax.axis_index("c")
    s = lax.axis_index("s")
    pltpu.sync_copy(zeros_hbm.at[pl.ds(s * RPT, RPT)],
                    deg_sh.at[pl.ds(s * RPT, RPT)])
    for b in range(NBLK):
        pltpu.sync_copy(ed_hbm.at[c, s, b, 1], dst_v.at[b])
    pltpu.sync_copy(ones_hbm, ones_v)
    plsc.subcore_barrier()

    def body(b, carry):
        def inner(j, carry2):
            pltpu.sync_copy(ones_v, deg_sh.at[dst_v.at[b, j]], add=True)
            return carry2
        return lax.fori_loop(0, IB, inner, carry)

    lax.fori_loop(0, NBLK, body, 0)
    plsc.subcore_barrier()
    pltpu.sync_copy(deg_sh.at[pl.ds(s * RPT, RPT)],
                    out_hbm.at[c, pl.ds(s * RPT, RPT)])


_sc_deg = functools.partial(
    pl.kernel,
    out_type=jax.ShapeDtypeStruct((NC, NP, D), jnp.float32),
    mesh=_mesh,
    scratch_types=[
        pltpu.VMEM((NBLK, IB, CHUNK), jnp.int32),
        pltpu.VMEM((CHUNK, D), jnp.float32),
        pltpu.VMEM_SHARED((NP, D), jnp.float32),
    ],
)(_sc_deg_body)


def _sc_agg_body(h_hbm, ed_hbm, zeros_hbm, out_hbm,
                 ed_v, rows_v, acc_sh, sem_i, sem_a, sem_b):
    c = lax.axis_index("c")
    s = lax.axis_index("s")
    pltpu.sync_copy(zeros_hbm.at[pl.ds(s * RPT, RPT)],
                    acc_sh.at[pl.ds(s * RPT, RPT)])
    pltpu.sync_copy(ed_hbm.at[c, s, 0], ed_v.at[0])
    plsc.subcore_barrier()

    def blk(b, carry):
        pb = lax.rem(b, 2)
        nb = 1 - pb

        @pl.when(b < NBLK - 1)
        def _prefetch_idx():
            pltpu.async_copy(ed_hbm.at[c, s, b + 1], ed_v.at[nb], sem_i)

        pltpu.async_copy(h_hbm.at[ed_v.at[pb, 0, 0]], rows_v.at[0], sem_a)

        def pair(g, carry2):
            j0 = 2 * g
            j1 = j0 + 1
            pltpu.async_copy(h_hbm.at[ed_v.at[pb, 0, j1]], rows_v.at[1], sem_b)
            pltpu.make_async_copy(h_hbm.at[ed_v.at[pb, 0, j0]],
                                  rows_v.at[0], sem_a).wait()
            pltpu.sync_copy(rows_v.at[0], acc_sh.at[ed_v.at[pb, 1, j0]],
                            add=True)

            @pl.when(g < IB // 2 - 1)
            def _prefetch_rows():
                pltpu.async_copy(h_hbm.at[ed_v.at[pb, 0, j0 + 2]],
                                 rows_v.at[0], sem_a)

            pltpu.make_async_copy(h_hbm.at[ed_v.at[pb, 0, j1]],
                                  rows_v.at[1], sem_b).wait()
            pltpu.sync_copy(rows_v.at[1], acc_sh.at[ed_v.at[pb, 1, j1]],
                            add=True)
            return carry2

        lax.fori_loop(0, IB // 2, pair, 0)

        @pl.when(b < NBLK - 1)
        def _wait_idx():
            pltpu.make_async_copy(ed_hbm.at[c, s, b + 1], ed_v.at[nb],
                                  sem_i).wait()

        return carry

    lax.fori_loop(0, NBLK, blk, 0)
    plsc.subcore_barrier()
    pltpu.sync_copy(acc_sh.at[pl.ds(s * RPT, RPT)],
                    out_hbm.at[c, pl.ds(s * RPT, RPT)])


_sc_agg = functools.partial(
    pl.kernel,
    out_type=jax.ShapeDtypeStruct((NC, NP, D), jnp.float32),
    mesh=_mesh,
    scratch_types=[
        pltpu.VMEM((2, 2, IB, CHUNK), jnp.int32),
        pltpu.VMEM((2, CHUNK, D), jnp.float32),
        pltpu.VMEM_SHARED((NP, D), jnp.float32),
        pltpu.SemaphoreType.DMA,
        pltpu.SemaphoreType.DMA,
        pltpu.SemaphoreType.DMA,
    ],
)(_sc_agg_body)


BR = 2048


def _dinv_of(deg_ref):
    deg = deg_ref[0, :, 0] + deg_ref[1, :, 0]
    return jnp.where(deg > 0, lax.rsqrt(deg), 0.0)


def _tc_pre_body(x_ref, w_ref, deg_ref, o_ref):
    dinv = _dinv_of(deg_ref)
    h = jnp.dot(x_ref[...], w_ref[...], preferred_element_type=jnp.float32)
    o_ref[...] = h * dinv[:, None]


def _tc_mid_body(acc_ref, deg_ref, b_ref, w_ref, o_ref):
    dinv = _dinv_of(deg_ref)
    t = (acc_ref[0] + acc_ref[1]) * dinv[:, None] + b_ref[...]
    r = jnp.maximum(t, 0.0)
    o_ref[...] = jnp.dot(r, w_ref[...],
                         preferred_element_type=jnp.float32) * dinv[:, None]


def _tc_post_body(acc_ref, deg_ref, b_ref, o_ref):
    dinv = _dinv_of(deg_ref)
    o_ref[...] = (acc_ref[0] + acc_ref[1]) * dinv[:, None] + b_ref[...]


_acc_spec = pl.BlockSpec((NC, BR, D), lambda i: (0, i, 0))
_deg_spec = _acc_spec
_row_spec = pl.BlockSpec((BR, D), lambda i: (i, 0))
_w_spec = pl.BlockSpec((D, D), lambda i: (0, 0))
_b_spec = pl.BlockSpec((1, D), lambda i: (0, 0))

_tc_pre = pl.pallas_call(
    _tc_pre_body,
    grid=(NP // BR,),
    in_specs=[_row_spec, _w_spec, _deg_spec],
    out_specs=_row_spec,
    out_shape=jax.ShapeDtypeStruct((NP, D), jnp.float32),
)

_tc_mid = pl.pallas_call(
    _tc_mid_body,
    grid=(NP // BR,),
    in_specs=[_acc_spec, _deg_spec, _b_spec, _w_spec],
    out_specs=_row_spec,
    out_shape=jax.ShapeDtypeStruct((NP, D), jnp.float32),
)

_tc_post = pl.pallas_call(
    _tc_post_body,
    grid=(NP // BR,),
    in_specs=[_acc_spec, _deg_spec, _b_spec],
    out_specs=_row_spec,
    out_shape=jax.ShapeDtypeStruct((NP, D), jnp.float32),
)


def kernel(x, edge_index, W1, b1, W2, b2):
    loop = jnp.arange(N, dtype=jnp.int32)
    pad = E_PAD - (E + N)
    src = jnp.concatenate([
        edge_index[0].astype(jnp.int32), loop,
        jnp.zeros((pad,), jnp.int32),
    ]).reshape(NC, NS, NBLK, 1, IB, CHUNK)
    dst = jnp.concatenate([
        edge_index[1].astype(jnp.int32), loop,
        jnp.full((pad,), N, jnp.int32),
    ]).reshape(NC, NS, NBLK, 1, IB, CHUNK)
    ed = jnp.concatenate([src, dst], axis=3)

    x_pad = jnp.pad(x, ((0, NP - N), (0, 0)))
    zeros_d = jnp.zeros((NP, D), jnp.float32)
    ones_d = jnp.ones((NP, D), jnp.float32)
    b1r = b1.reshape(1, D)
    b2r = b2.reshape(1, D)

    ones_chunk = jnp.ones((CHUNK, D), jnp.float32)
    deg_parts = _sc_deg(ed, ones_chunk, zeros_d)
    h1 = _tc_pre(x_pad, W1, deg_parts)
    acc1 = _sc_agg(h1, ed, zeros_d)
    h2 = _tc_mid(acc1, deg_parts, b1r, W2)
    acc2 = _sc_agg(h2, ed, zeros_d)
    out = _tc_post(acc2, deg_parts, b2r)
    return out[:N]


# speedup vs baseline: 4.0976x; 4.0976x over previous
import functools

import jax
import jax.numpy as jnp
from jax import lax
from jax.experimental import pallas as pl
from jax.experimental.pallas import tpu as pltpu
from jax.experimental.pallas import tpu_sc as plsc

N = 10000
NP = 10240
D = 128
E = 320000
NC = 2
NS = 16
CHUNK = 128
IB = 14
NBLK = 6
CHUNKS = NBLK * IB
E_PAD = NC * NS * CHUNKS * CHUNK
RPT = NP // NS

_mesh = plsc.VectorSubcoreMesh(core_axis_name="c", subcore_axis_name="s")


def _sc_deg_body(ed_hbm, ones_hbm, zeros_hbm, out_hbm, dst_v, ones_v, deg_sh):
    c = lax.axis_index("c")
    s = lax.axis_index("s")
    pltpu.sync_copy(zeros_hbm.at[pl.ds(s * RPT, RPT)],
                    deg_sh.at[pl.ds(s * RPT, RPT)])
    for b in range(NBLK):
        pltpu.sync_copy(ed_hbm.at[c, s, b, 1], dst_v.at[b])
    pltpu.sync_copy(ones_hbm, ones_v)
    plsc.subcore_barrier()

    def body(b, carry):
        def inner(j, carry2):
            pltpu.sync_copy(ones_v, deg_sh.at[dst_v.at[b, j]], add=True)
            return carry2
        return lax.fori_loop(0, IB, inner, carry)

    lax.fori_loop(0, NBLK, body, 0)
    plsc.subcore_barrier()
    pltpu.sync_copy(deg_sh.at[pl.ds(s * RPT, RPT)],
                    out_hbm.at[c, pl.ds(s * RPT, RPT)])


_sc_deg = functools.partial(
    pl.kernel,
    out_type=jax.ShapeDtypeStruct((NC, NP, D), jnp.float32),
    mesh=_mesh,
    scratch_types=[
        pltpu.VMEM((NBLK, IB, CHUNK), jnp.int32),
        pltpu.VMEM((CHUNK, D), jnp.float32),
        pltpu.VMEM_SHARED((NP, D), jnp.float32),
    ],
)(_sc_deg_body)


def _sc_agg_body(h_hbm, ed_hbm, zeros_hbm, out_hbm,
                 ed_v, rows_v, acc_sh, sem_i, sem_a, sem_b):
    c = lax.axis_index("c")
    s = lax.axis_index("s")
    pltpu.sync_copy(zeros_hbm.at[pl.ds(s * RPT, RPT)],
                    acc_sh.at[pl.ds(s * RPT, RPT)])
    pltpu.sync_copy(ed_hbm.at[c, s, 0], ed_v.at[0])
    plsc.subcore_barrier()

    def blk(b, carry):
        pb = lax.rem(b, 2)
        nb = 1 - pb

        @pl.when(b < NBLK - 1)
        def _prefetch_idx():
            pltpu.async_copy(ed_hbm.at[c, s, b + 1], ed_v.at[nb], sem_i)

        pltpu.async_copy(h_hbm.at[ed_v.at[pb, 0, 0]], rows_v.at[0], sem_a)

        def pair(g, carry2):
            j0 = 2 * g
            j1 = j0 + 1
            pltpu.async_copy(h_hbm.at[ed_v.at[pb, 0, j1]], rows_v.at[1], sem_b)
            pltpu.make_async_copy(h_hbm.at[ed_v.at[pb, 0, j0]],
                                  rows_v.at[0], sem_a).wait()
            pltpu.sync_copy(rows_v.at[0], acc_sh.at[ed_v.at[pb, 1, j0]],
                            add=True)

            @pl.when(g < IB // 2 - 1)
            def _prefetch_rows():
                pltpu.async_copy(h_hbm.at[ed_v.at[pb, 0, j0 + 2]],
                                 rows_v.at[0], sem_a)

            pltpu.make_async_copy(h_hbm.at[ed_v.at[pb, 0, j1]],
                                  rows_v.at[1], sem_b).wait()
            pltpu.sync_copy(rows_v.at[1], acc_sh.at[ed_v.at[pb, 1, j1]],
                            add=True)
            return carry2

        lax.fori_loop(0, IB // 2, pair, 0)

        @pl.when(b < NBLK - 1)
        def _wait_idx():
            pltpu.make_async_copy(ed_hbm.at[c, s, b + 1], ed_v.at[nb],
                                  sem_i).wait()

        return carry

    lax.fori_loop(0, NBLK, blk, 0)
    plsc.subcore_barrier()
    pltpu.sync_copy(acc_sh.at[pl.ds(s * RPT, RPT)],
                    out_hbm.at[c, pl.ds(s * RPT, RPT)])


_sc_agg = functools.partial(
    pl.kernel,
    out_type=jax.ShapeDtypeStruct((NC, NP, D), jnp.float32),
    mesh=_mesh,
    scratch_types=[
        pltpu.VMEM((2, 2, IB, CHUNK), jnp.int32),
        pltpu.VMEM((2, CHUNK, D), jnp.float32),
        pltpu.VMEM_SHARED((NP, D), jnp.float32),
        pltpu.SemaphoreType.DMA,
        pltpu.SemaphoreType.DMA,
        pltpu.SemaphoreType.DMA,
    ],
)(_sc_agg_body)


BR = 2048


def _dinv_of(deg_ref):
    deg = deg_ref[0, :, 0] + deg_ref[1, :, 0]
    return jnp.where(deg > 0, lax.rsqrt(deg), 0.0)


def _tc_pre_body(x_ref, w_ref, deg_ref, o_ref):
    dinv = _dinv_of(deg_ref)
    h = jnp.dot(x_ref[...], w_ref[...], preferred_element_type=jnp.float32)
    o_ref[...] = h * dinv[:, None]


def _tc_mid_body(acc_ref, deg_ref, b_ref, w_ref, o_ref):
    dinv = _dinv_of(deg_ref)
    t = (acc_ref[0] + acc_ref[1]) * dinv[:, None] + b_ref[...]
    r = jnp.maximum(t, 0.0)
    o_ref[...] = jnp.dot(r, w_ref[...],
                         preferred_element_type=jnp.float32) * dinv[:, None]


def _tc_post_body(acc_ref, deg_ref, b_ref, o_ref):
    dinv = _dinv_of(deg_ref)
    o_ref[...] = (acc_ref[0] + acc_ref[1]) * dinv[:, None] + b_ref[...]


_acc_spec = pl.BlockSpec((NC, BR, D), lambda i: (0, i, 0))
_deg_spec = _acc_spec
_row_spec = pl.BlockSpec((BR, D), lambda i: (i, 0))
_w_spec = pl.BlockSpec((D, D), lambda i: (0, 0))
_b_spec = pl.BlockSpec((1, D), lambda i: (0, 0))

_tc_pre = pl.pallas_call(
    _tc_pre_body,
    grid=(NP // BR,),
    in_specs=[_row_spec, _w_spec, _deg_spec],
    out_specs=_row_spec,
    out_shape=jax.ShapeDtypeStruct((NP, D), jnp.float32),
)

_tc_mid = pl.pallas_call(
    _tc_mid_body,
    grid=(NP // BR,),
    in_specs=[_acc_spec, _deg_spec, _b_spec, _w_spec],
    out_specs=_row_spec,
    out_shape=jax.ShapeDtypeStruct((NP, D), jnp.float32),
)

_tc_post = pl.pallas_call(
    _tc_post_body,
    grid=(NP // BR,),
    in_specs=[_acc_spec, _deg_spec, _b_spec],
    out_specs=_row_spec,
    out_shape=jax.ShapeDtypeStruct((NP, D), jnp.float32),
)


def kernel(x, edge_index, W1, b1, W2, b2):
    loop = jnp.arange(N, dtype=jnp.int32)
    pad = E_PAD - (E + N)
    pad_ar = jnp.arange(pad, dtype=jnp.int32)
    src = jnp.concatenate([
        edge_index[0].astype(jnp.int32), loop,
        pad_ar % N,
    ]).reshape(NC, NS, NBLK, 1, IB, CHUNK)
    dst = jnp.concatenate([
        edge_index[1].astype(jnp.int32), loop,
        N + pad_ar % (NP - N),
    ]).reshape(NC, NS, NBLK, 1, IB, CHUNK)
    ed = jnp.concatenate([src, dst], axis=3)

    x_pad = jnp.pad(x, ((0, NP - N), (0, 0)))
    zeros_d = jnp.zeros((NP, D), jnp.float32)
    ones_d = jnp.ones((NP, D), jnp.float32)
    b1r = b1.reshape(1, D)
    b2r = b2.reshape(1, D)

    ones_chunk = jnp.ones((CHUNK, D), jnp.float32)
    deg_parts = _sc_deg(ed, ones_chunk, zeros_d)
    h1 = _tc_pre(x_pad, W1, deg_parts)
    acc1 = _sc_agg(h1, ed, zeros_d)
    h2 = _tc_mid(acc1, deg_parts, b1r, W2)
    acc2 = _sc_agg(h2, ed, zeros_d)
    out = _tc_post(acc2, deg_parts, b2r)
    return out[:N]


# speedup vs baseline: 4.1118x; 1.0035x over previous
import functools

import jax
import jax.numpy as jnp
from jax import lax
from jax.experimental import pallas as pl
from jax.experimental.pallas import tpu as pltpu
from jax.experimental.pallas import tpu_sc as plsc

N = 10000
NP = 10240
D = 128
E = 320000
NC = 2
NS = 16
CHUNK = 128
IB = 14
NBLK = 6
CHUNKS = NBLK * IB
E_PAD = NC * NS * CHUNKS * CHUNK
RPT = NP // NS

_mesh = plsc.VectorSubcoreMesh(core_axis_name="c", subcore_axis_name="s")


def _sc_deg_body(ed_hbm, ones_hbm, zeros_hbm, out_hbm, dst_v, ones_v, deg_sh,
                 sem):
    c = lax.axis_index("c")
    s = lax.axis_index("s")
    pltpu.sync_copy(zeros_hbm.at[pl.ds(s * RPT, RPT)],
                    deg_sh.at[pl.ds(s * RPT, RPT)])
    for b in range(NBLK):
        pltpu.sync_copy(ed_hbm.at[c, s, b, 1], dst_v.at[b])
    pltpu.sync_copy(ones_hbm, ones_v)
    plsc.subcore_barrier()

    def body(b, carry):
        def fire(j, carry2):
            pltpu.async_copy(ones_v, deg_sh.at[dst_v.at[b, j]], sem,
                             add=True)
            return carry2

        lax.fori_loop(0, IB, fire, carry)

        def drain(j, carry2):
            pltpu.make_async_copy(ones_v, deg_sh.at[dst_v.at[b, j]],
                                  sem).wait()
            return carry2

        return lax.fori_loop(0, IB, drain, carry)

    lax.fori_loop(0, NBLK, body, 0)
    plsc.subcore_barrier()
    pltpu.sync_copy(deg_sh.at[pl.ds(s * RPT, RPT)],
                    out_hbm.at[c, pl.ds(s * RPT, RPT)])


_sc_deg = functools.partial(
    pl.kernel,
    out_type=jax.ShapeDtypeStruct((NC, NP, D), jnp.float32),
    mesh=_mesh,
    scratch_types=[
        pltpu.VMEM((NBLK, IB, CHUNK), jnp.int32),
        pltpu.VMEM((CHUNK, D), jnp.float32),
        pltpu.VMEM_SHARED((NP, D), jnp.float32),
        pltpu.SemaphoreType.DMA,
    ],
)(_sc_deg_body)


def _sc_agg_body(h_hbm, ed_hbm, zeros_hbm, out_hbm,
                 ed_v, rows_v, acc_sh, sem_i, sem_a, sem_b, sem_sa, sem_sb):
    c = lax.axis_index("c")
    s = lax.axis_index("s")
    pltpu.sync_copy(zeros_hbm.at[pl.ds(s * RPT, RPT)],
                    acc_sh.at[pl.ds(s * RPT, RPT)])
    pltpu.sync_copy(ed_hbm.at[c, s, 0], ed_v.at[0])
    plsc.subcore_barrier()

    def blk(b, carry):
        pb = lax.rem(b, 2)
        nb = 1 - pb

        @pl.when(b < NBLK - 1)
        def _prefetch_idx():
            pltpu.async_copy(ed_hbm.at[c, s, b + 1], ed_v.at[nb], sem_i)

        pltpu.async_copy(h_hbm.at[ed_v.at[pb, 0, 0]], rows_v.at[0], sem_a)

        def pair(g, carry2):
            j0 = 2 * g
            j1 = j0 + 1
            pltpu.async_copy(h_hbm.at[ed_v.at[pb, 0, j1]], rows_v.at[1], sem_b)
            pltpu.make_async_copy(h_hbm.at[ed_v.at[pb, 0, j0]],
                                  rows_v.at[0], sem_a).wait()
            pltpu.async_copy(rows_v.at[0], acc_sh.at[ed_v.at[pb, 1, j0]],
                             sem_sa, add=True)

            @pl.when(g < IB // 2 - 1)
            def _reuse_buf0():
                pltpu.make_async_copy(rows_v.at[0],
                                      acc_sh.at[ed_v.at[pb, 1, j0]],
                                      sem_sa).wait()
                pltpu.async_copy(h_hbm.at[ed_v.at[pb, 0, j0 + 2]],
                                 rows_v.at[0], sem_a)

            pltpu.make_async_copy(h_hbm.at[ed_v.at[pb, 0, j1]],
                                  rows_v.at[1], sem_b).wait()
            pltpu.async_copy(rows_v.at[1], acc_sh.at[ed_v.at[pb, 1, j1]],
                             sem_sb, add=True)

            @pl.when(g < IB // 2 - 1)
            def _drain_buf1():
                pltpu.make_async_copy(rows_v.at[1],
                                      acc_sh.at[ed_v.at[pb, 1, j1]],
                                      sem_sb).wait()
            return carry2

        lax.fori_loop(0, IB // 2, pair, 0)
        pltpu.make_async_copy(rows_v.at[0], acc_sh.at[ed_v.at[pb, 1, IB - 2]],
                              sem_sa).wait()
        pltpu.make_async_copy(rows_v.at[1], acc_sh.at[ed_v.at[pb, 1, IB - 1]],
                              sem_sb).wait()

        @pl.when(b < NBLK - 1)
        def _wait_idx():
            pltpu.make_async_copy(ed_hbm.at[c, s, b + 1], ed_v.at[nb],
                                  sem_i).wait()

        return carry

    lax.fori_loop(0, NBLK, blk, 0)
    plsc.subcore_barrier()
    pltpu.sync_copy(acc_sh.at[pl.ds(s * RPT, RPT)],
                    out_hbm.at[c, pl.ds(s * RPT, RPT)])


_sc_agg = functools.partial(
    pl.kernel,
    out_type=jax.ShapeDtypeStruct((NC, NP, D), jnp.float32),
    mesh=_mesh,
    scratch_types=[
        pltpu.VMEM((2, 2, IB, CHUNK), jnp.int32),
        pltpu.VMEM((2, CHUNK, D), jnp.float32),
        pltpu.VMEM_SHARED((NP, D), jnp.float32),
        pltpu.SemaphoreType.DMA,
        pltpu.SemaphoreType.DMA,
        pltpu.SemaphoreType.DMA,
        pltpu.SemaphoreType.DMA,
        pltpu.SemaphoreType.DMA,
    ],
)(_sc_agg_body)


BR = 2048


def _dinv_of(deg_ref):
    deg = deg_ref[0, :, 0] + deg_ref[1, :, 0]
    return jnp.where(deg > 0, lax.rsqrt(deg), 0.0)


def _tc_pre_body(x_ref, w_ref, deg_ref, o_ref):
    dinv = _dinv_of(deg_ref)
    h = jnp.dot(x_ref[...], w_ref[...], preferred_element_type=jnp.float32)
    o_ref[...] = h * dinv[:, None]


def _tc_mid_body(acc_ref, deg_ref, b_ref, w_ref, o_ref):
    dinv = _dinv_of(deg_ref)
    t = (acc_ref[0] + acc_ref[1]) * dinv[:, None] + b_ref[...]
    r = jnp.maximum(t, 0.0)
    o_ref[...] = jnp.dot(r, w_ref[...],
                         preferred_element_type=jnp.float32) * dinv[:, None]


def _tc_post_body(acc_ref, deg_ref, b_ref, o_ref):
    dinv = _dinv_of(deg_ref)
    o_ref[...] = (acc_ref[0] + acc_ref[1]) * dinv[:, None] + b_ref[...]


_acc_spec = pl.BlockSpec((NC, BR, D), lambda i: (0, i, 0))
_deg_spec = _acc_spec
_row_spec = pl.BlockSpec((BR, D), lambda i: (i, 0))
_w_spec = pl.BlockSpec((D, D), lambda i: (0, 0))
_b_spec = pl.BlockSpec((1, D), lambda i: (0, 0))

_tc_pre = pl.pallas_call(
    _tc_pre_body,
    grid=(NP // BR,),
    in_specs=[_row_spec, _w_spec, _deg_spec],
    out_specs=_row_spec,
    out_shape=jax.ShapeDtypeStruct((NP, D), jnp.float32),
)

_tc_mid = pl.pallas_call(
    _tc_mid_body,
    grid=(NP // BR,),
    in_specs=[_acc_spec, _deg_spec, _b_spec, _w_spec],
    out_specs=_row_spec,
    out_shape=jax.ShapeDtypeStruct((NP, D), jnp.float32),
)

_tc_post = pl.pallas_call(
    _tc_post_body,
    grid=(NP // BR,),
    in_specs=[_acc_spec, _deg_spec, _b_spec],
    out_specs=_row_spec,
    out_shape=jax.ShapeDtypeStruct((NP, D), jnp.float32),
)


def kernel(x, edge_index, W1, b1, W2, b2):
    loop = jnp.arange(N, dtype=jnp.int32)
    pad = E_PAD - (E + N)
    pad_ar = jnp.arange(pad, dtype=jnp.int32)
    src = jnp.concatenate([
        edge_index[0].astype(jnp.int32), loop,
        pad_ar % N,
    ]).reshape(NC, NS, NBLK, 1, IB, CHUNK)
    dst = jnp.concatenate([
        edge_index[1].astype(jnp.int32), loop,
        N + pad_ar % (NP - N),
    ]).reshape(NC, NS, NBLK, 1, IB, CHUNK)
    ed = jnp.concatenate([src, dst], axis=3)

    x_pad = jnp.pad(x, ((0, NP - N), (0, 0)))
    zeros_d = jnp.zeros((NP, D), jnp.float32)
    ones_d = jnp.ones((NP, D), jnp.float32)
    b1r = b1.reshape(1, D)
    b2r = b2.reshape(1, D)

    ones_chunk = jnp.ones((CHUNK, D), jnp.float32)
    deg_parts = _sc_deg(ed, ones_chunk, zeros_d)
    h1 = _tc_pre(x_pad, W1, deg_parts)
    acc1 = _sc_agg(h1, ed, zeros_d)
    h2 = _tc_mid(acc1, deg_parts, b1r, W2)
    acc2 = _sc_agg(h2, ed, zeros_d)
    out = _tc_post(acc2, deg_parts, b2r)
    return out[:N]


# speedup vs baseline: 4.4048x; 1.0712x over previous
import functools

import jax
import jax.numpy as jnp
from jax import lax
from jax.experimental import pallas as pl
from jax.experimental.pallas import tpu as pltpu
from jax.experimental.pallas import tpu_sc as plsc

N = 10000
NP = 10240
D = 128
E = 320000
NC = 2
NS = 16
CHUNK = 128
IB = 16
NBLK = 5
CHUNKS = NBLK * IB
E_PAD = NC * NS * CHUNKS * CHUNK
RPT = NP // NS

_mesh = plsc.VectorSubcoreMesh(core_axis_name="c", subcore_axis_name="s")


def _sc_deg_body(ed_hbm, ones_hbm, onesd_hbm, zeros_hbm, out_hbm,
                 dst_v, ones_v, deg_sh, sem):
    c = lax.axis_index("c")
    s = lax.axis_index("s")

    @pl.when(c == 0)
    def _init_ones():
        pltpu.sync_copy(onesd_hbm.at[pl.ds(s * RPT, RPT)],
                        deg_sh.at[pl.ds(s * RPT, RPT)])

    @pl.when(c != 0)
    def _init_zeros():
        pltpu.sync_copy(zeros_hbm.at[pl.ds(s * RPT, RPT)],
                        deg_sh.at[pl.ds(s * RPT, RPT)])
    for b in range(NBLK):
        pltpu.sync_copy(ed_hbm.at[c, s, b, 1], dst_v.at[b])
    pltpu.sync_copy(ones_hbm, ones_v)
    plsc.subcore_barrier()

    def body(b, carry):
        def fire(j, carry2):
            pltpu.async_copy(ones_v, deg_sh.at[dst_v.at[b, j]], sem,
                             add=True)
            return carry2

        lax.fori_loop(0, IB, fire, carry)

        def drain(j, carry2):
            pltpu.make_async_copy(ones_v, deg_sh.at[dst_v.at[b, j]],
                                  sem).wait()
            return carry2

        return lax.fori_loop(0, IB, drain, carry)

    lax.fori_loop(0, NBLK, body, 0)
    plsc.subcore_barrier()
    pltpu.sync_copy(deg_sh.at[pl.ds(s * RPT, RPT)],
                    out_hbm.at[c, pl.ds(s * RPT, RPT)])


_sc_deg = functools.partial(
    pl.kernel,
    out_type=jax.ShapeDtypeStruct((NC, NP, D), jnp.float32),
    mesh=_mesh,
    scratch_types=[
        pltpu.VMEM((NBLK, IB, CHUNK), jnp.int32),
        pltpu.VMEM((CHUNK, D), jnp.float32),
        pltpu.VMEM_SHARED((NP, D), jnp.float32),
        pltpu.SemaphoreType.DMA,
    ],
)(_sc_deg_body)


def _sc_agg_body(h_hbm, ed_hbm, zeros_hbm, out_hbm,
                 ed_v, rows_v, acc_sh, sem_i, sem_a, sem_b, sem_sa, sem_sb):
    c = lax.axis_index("c")
    s = lax.axis_index("s")

    @pl.when(c == 0)
    def _init_h():
        pltpu.sync_copy(h_hbm.at[pl.ds(s * RPT, RPT)],
                        acc_sh.at[pl.ds(s * RPT, RPT)])

    @pl.when(c != 0)
    def _init_zeros():
        pltpu.sync_copy(zeros_hbm.at[pl.ds(s * RPT, RPT)],
                        acc_sh.at[pl.ds(s * RPT, RPT)])
    pltpu.sync_copy(ed_hbm.at[c, s, 0], ed_v.at[0])
    plsc.subcore_barrier()

    def blk(b, carry):
        pb = lax.rem(b, 2)
        nb = 1 - pb

        @pl.when(b < NBLK - 1)
        def _prefetch_idx():
            pltpu.async_copy(ed_hbm.at[c, s, b + 1], ed_v.at[nb], sem_i)

        pltpu.async_copy(h_hbm.at[ed_v.at[pb, 0, 0]], rows_v.at[0], sem_a)

        def pair(g, carry2):
            j0 = 2 * g
            j1 = j0 + 1
            pltpu.async_copy(h_hbm.at[ed_v.at[pb, 0, j1]], rows_v.at[1], sem_b)
            pltpu.make_async_copy(h_hbm.at[ed_v.at[pb, 0, j0]],
                                  rows_v.at[0], sem_a).wait()
            pltpu.async_copy(rows_v.at[0], acc_sh.at[ed_v.at[pb, 1, j0]],
                             sem_sa, add=True)

            @pl.when(g < IB // 2 - 1)
            def _reuse_buf0():
                pltpu.make_async_copy(rows_v.at[0],
                                      acc_sh.at[ed_v.at[pb, 1, j0]],
                                      sem_sa).wait()
                pltpu.async_copy(h_hbm.at[ed_v.at[pb, 0, j0 + 2]],
                                 rows_v.at[0], sem_a)

            pltpu.make_async_copy(h_hbm.at[ed_v.at[pb, 0, j1]],
                                  rows_v.at[1], sem_b).wait()
            pltpu.async_copy(rows_v.at[1], acc_sh.at[ed_v.at[pb, 1, j1]],
                             sem_sb, add=True)

            @pl.when(g < IB // 2 - 1)
            def _drain_buf1():
                pltpu.make_async_copy(rows_v.at[1],
                                      acc_sh.at[ed_v.at[pb, 1, j1]],
                                      sem_sb).wait()
            return carry2

        lax.fori_loop(0, IB // 2, pair, 0)
        pltpu.make_async_copy(rows_v.at[0], acc_sh.at[ed_v.at[pb, 1, IB - 2]],
                              sem_sa).wait()
        pltpu.make_async_copy(rows_v.at[1], acc_sh.at[ed_v.at[pb, 1, IB - 1]],
                              sem_sb).wait()

        @pl.when(b < NBLK - 1)
        def _wait_idx():
            pltpu.make_async_copy(ed_hbm.at[c, s, b + 1], ed_v.at[nb],
                                  sem_i).wait()

        return carry

    lax.fori_loop(0, NBLK, blk, 0)
    plsc.subcore_barrier()
    pltpu.sync_copy(acc_sh.at[pl.ds(s * RPT, RPT)],
                    out_hbm.at[c, pl.ds(s * RPT, RPT)])


_sc_agg = functools.partial(
    pl.kernel,
    out_type=jax.ShapeDtypeStruct((NC, NP, D), jnp.float32),
    mesh=_mesh,
    scratch_types=[
        pltpu.VMEM((2, 2, IB, CHUNK), jnp.int32),
        pltpu.VMEM((2, CHUNK, D), jnp.float32),
        pltpu.VMEM_SHARED((NP, D), jnp.float32),
        pltpu.SemaphoreType.DMA,
        pltpu.SemaphoreType.DMA,
        pltpu.SemaphoreType.DMA,
        pltpu.SemaphoreType.DMA,
        pltpu.SemaphoreType.DMA,
    ],
)(_sc_agg_body)


BR = 2048


def _dinv_of(deg_ref):
    deg = deg_ref[0, :, 0] + deg_ref[1, :, 0]
    return jnp.where(deg > 0, lax.rsqrt(deg), 0.0)


def _tc_pre_body(x_ref, w_ref, deg_ref, o_ref):
    dinv = _dinv_of(deg_ref)
    h = jnp.dot(x_ref[...], w_ref[...], preferred_element_type=jnp.float32)
    o_ref[...] = h * dinv[:, None]


def _tc_mid_body(acc_ref, deg_ref, b_ref, w_ref, o_ref):
    dinv = _dinv_of(deg_ref)
    t = (acc_ref[0] + acc_ref[1]) * dinv[:, None] + b_ref[...]
    r = jnp.maximum(t, 0.0)
    o_ref[...] = jnp.dot(r, w_ref[...],
                         preferred_element_type=jnp.float32) * dinv[:, None]


def _tc_post_body(acc_ref, deg_ref, b_ref, o_ref):
    dinv = _dinv_of(deg_ref)
    o_ref[...] = (acc_ref[0] + acc_ref[1]) * dinv[:, None] + b_ref[...]


_acc_spec = pl.BlockSpec((NC, BR, D), lambda i: (0, i, 0))
_deg_spec = _acc_spec
_row_spec = pl.BlockSpec((BR, D), lambda i: (i, 0))
_w_spec = pl.BlockSpec((D, D), lambda i: (0, 0))
_b_spec = pl.BlockSpec((1, D), lambda i: (0, 0))

_tc_pre = pl.pallas_call(
    _tc_pre_body,
    grid=(NP // BR,),
    in_specs=[_row_spec, _w_spec, _deg_spec],
    out_specs=_row_spec,
    out_shape=jax.ShapeDtypeStruct((NP, D), jnp.float32),
)

_tc_mid = pl.pallas_call(
    _tc_mid_body,
    grid=(NP // BR,),
    in_specs=[_acc_spec, _deg_spec, _b_spec, _w_spec],
    out_specs=_row_spec,
    out_shape=jax.ShapeDtypeStruct((NP, D), jnp.float32),
)

_tc_post = pl.pallas_call(
    _tc_post_body,
    grid=(NP // BR,),
    in_specs=[_acc_spec, _deg_spec, _b_spec],
    out_specs=_row_spec,
    out_shape=jax.ShapeDtypeStruct((NP, D), jnp.float32),
)


def kernel(x, edge_index, W1, b1, W2, b2):
    pad = E_PAD - E
    pad_ar = jnp.arange(pad, dtype=jnp.int32)
    src = jnp.concatenate([
        edge_index[0].astype(jnp.int32),
        pad_ar % N,
    ]).reshape(NC, NS, NBLK, 1, IB, CHUNK)
    dst = jnp.concatenate([
        edge_index[1].astype(jnp.int32),
        N + pad_ar % (NP - N),
    ]).reshape(NC, NS, NBLK, 1, IB, CHUNK)
    ed = jnp.concatenate([src, dst], axis=3)

    x_pad = jnp.pad(x, ((0, NP - N), (0, 0)))
    zeros_d = jnp.zeros((NP, D), jnp.float32)
    ones_d = jnp.ones((NP, D), jnp.float32)
    b1r = b1.reshape(1, D)
    b2r = b2.reshape(1, D)

    ones_chunk = jnp.ones((CHUNK, D), jnp.float32)
    deg_parts = _sc_deg(ed, ones_chunk, ones_d, zeros_d)
    h1 = _tc_pre(x_pad, W1, deg_parts)
    acc1 = _sc_agg(h1, ed, zeros_d)
    h2 = _tc_mid(acc1, deg_parts, b1r, W2)
    acc2 = _sc_agg(h2, ed, zeros_d)
    out = _tc_post(acc2, deg_parts, b2r)
    return out[:N]


# speedup vs baseline: 4.4154x; 1.0024x over previous
import functools

import jax
import jax.numpy as jnp
from jax import lax
from jax.experimental import pallas as pl
from jax.experimental.pallas import tpu as pltpu
from jax.experimental.pallas import tpu_sc as plsc

N = 10000
NP = 10240
D = 128
E = 320000
NC = 2
NS = 16
CHUNK = 128
IB = 16
NBLK = 5
CHUNKS = NBLK * IB
E_PAD = NC * NS * CHUNKS * CHUNK
RPT = NP // NS

_mesh = plsc.VectorSubcoreMesh(core_axis_name="c", subcore_axis_name="s")


def _sc_deg_body(ed_hbm, ones_hbm, onesd_hbm, zeros_hbm, out_hbm,
                 dst_v, ones_v, deg_sh, sem):
    c = lax.axis_index("c")
    s = lax.axis_index("s")

    @pl.when(c == 0)
    def _init_ones():
        pltpu.sync_copy(onesd_hbm.at[pl.ds(s * RPT, RPT)],
                        deg_sh.at[pl.ds(s * RPT, RPT)])

    @pl.when(c != 0)
    def _init_zeros():
        pltpu.sync_copy(zeros_hbm.at[pl.ds(s * RPT, RPT)],
                        deg_sh.at[pl.ds(s * RPT, RPT)])
    for b in range(NBLK):
        pltpu.sync_copy(ed_hbm.at[c, s, b, 1], dst_v.at[b])
    pltpu.sync_copy(ones_hbm, ones_v)
    plsc.subcore_barrier()

    def body(b, carry):
        def fire(j, carry2):
            pltpu.async_copy(ones_v, deg_sh.at[dst_v.at[b, j]], sem,
                             add=True)
            return carry2

        lax.fori_loop(0, IB, fire, carry)

        def drain(j, carry2):
            pltpu.make_async_copy(ones_v, deg_sh.at[dst_v.at[b, j]],
                                  sem).wait()
            return carry2

        return lax.fori_loop(0, IB, drain, carry)

    lax.fori_loop(0, NBLK, body, 0)
    plsc.subcore_barrier()
    pltpu.sync_copy(deg_sh.at[pl.ds(s * RPT, RPT)],
                    out_hbm.at[c, pl.ds(s * RPT, RPT)])


_sc_deg = functools.partial(
    pl.kernel,
    out_type=jax.ShapeDtypeStruct((NC, NP, D), jnp.float32),
    mesh=_mesh,
    scratch_types=[
        pltpu.VMEM((NBLK, IB, CHUNK), jnp.int32),
        pltpu.VMEM((CHUNK, D), jnp.float32),
        pltpu.VMEM_SHARED((NP, D), jnp.float32),
        pltpu.SemaphoreType.DMA,
    ],
)(_sc_deg_body)


def _sc_agg_body(h_hbm, ed_hbm, zeros_hbm, out_hbm,
                 ed_v, rows_v, acc_sh, sem_i, sem_a, sem_b, sem_sa, sem_sb):
    c = lax.axis_index("c")
    s = lax.axis_index("s")

    @pl.when(c == 0)
    def _init_h():
        pltpu.sync_copy(h_hbm.at[pl.ds(s * RPT, RPT)],
                        acc_sh.at[pl.ds(s * RPT, RPT)])

    @pl.when(c != 0)
    def _init_zeros():
        pltpu.sync_copy(zeros_hbm.at[pl.ds(s * RPT, RPT)],
                        acc_sh.at[pl.ds(s * RPT, RPT)])
    pltpu.sync_copy(ed_hbm.at[c, s, 0], ed_v.at[0])
    plsc.subcore_barrier()

    def blk(b, carry):
        pb = lax.rem(b, 2)
        nb = 1 - pb

        @pl.when(b < NBLK - 1)
        def _prefetch_idx():
            pltpu.async_copy(ed_hbm.at[c, s, b + 1], ed_v.at[nb], sem_i)

        pltpu.async_copy(h_hbm.at[ed_v.at[pb, 0, 0, pl.ds(0, CHUNK // 2)]],
                         rows_v.at[0, pl.ds(0, CHUNK // 2)], sem_a)
        pltpu.async_copy(
            h_hbm.at[ed_v.at[pb, 0, 0, pl.ds(CHUNK // 2, CHUNK // 2)]],
            rows_v.at[0, pl.ds(CHUNK // 2, CHUNK // 2)], sem_a)

        H = CHUNK // 2

        def _gather(j, slot, sem):
            pltpu.async_copy(h_hbm.at[ed_v.at[pb, 0, j, pl.ds(0, H)]],
                             rows_v.at[slot, pl.ds(0, H)], sem)
            pltpu.async_copy(h_hbm.at[ed_v.at[pb, 0, j, pl.ds(H, H)]],
                             rows_v.at[slot, pl.ds(H, H)], sem)

        def _gwait(j, slot, sem):
            pltpu.make_async_copy(h_hbm.at[ed_v.at[pb, 0, j, pl.ds(0, H)]],
                                  rows_v.at[slot, pl.ds(0, H)], sem).wait()
            pltpu.make_async_copy(h_hbm.at[ed_v.at[pb, 0, j, pl.ds(H, H)]],
                                  rows_v.at[slot, pl.ds(H, H)], sem).wait()

        def pair(g, carry2):
            j0 = 2 * g
            j1 = j0 + 1
            _gather(j1, 1, sem_b)
            _gwait(j0, 0, sem_a)
            pltpu.async_copy(rows_v.at[0], acc_sh.at[ed_v.at[pb, 1, j0]],
                             sem_sa, add=True)

            @pl.when(g < IB // 2 - 1)
            def _reuse_buf0():
                pltpu.make_async_copy(rows_v.at[0],
                                      acc_sh.at[ed_v.at[pb, 1, j0]],
                                      sem_sa).wait()
                _gather(j0 + 2, 0, sem_a)

            _gwait(j1, 1, sem_b)
            pltpu.async_copy(rows_v.at[1], acc_sh.at[ed_v.at[pb, 1, j1]],
                             sem_sb, add=True)

            @pl.when(g < IB // 2 - 1)
            def _drain_buf1():
                pltpu.make_async_copy(rows_v.at[1],
                                      acc_sh.at[ed_v.at[pb, 1, j1]],
                                      sem_sb).wait()
            return carry2

        lax.fori_loop(0, IB // 2, pair, 0)
        pltpu.make_async_copy(rows_v.at[0], acc_sh.at[ed_v.at[pb, 1, IB - 2]],
                              sem_sa).wait()
        pltpu.make_async_copy(rows_v.at[1], acc_sh.at[ed_v.at[pb, 1, IB - 1]],
                              sem_sb).wait()

        @pl.when(b < NBLK - 1)
        def _wait_idx():
            pltpu.make_async_copy(ed_hbm.at[c, s, b + 1], ed_v.at[nb],
                                  sem_i).wait()

        return carry

    lax.fori_loop(0, NBLK, blk, 0)
    plsc.subcore_barrier()
    pltpu.sync_copy(acc_sh.at[pl.ds(s * RPT, RPT)],
                    out_hbm.at[c, pl.ds(s * RPT, RPT)])


_sc_agg = functools.partial(
    pl.kernel,
    out_type=jax.ShapeDtypeStruct((NC, NP, D), jnp.float32),
    mesh=_mesh,
    scratch_types=[
        pltpu.VMEM((2, 2, IB, CHUNK), jnp.int32),
        pltpu.VMEM((2, CHUNK, D), jnp.float32),
        pltpu.VMEM_SHARED((NP, D), jnp.float32),
        pltpu.SemaphoreType.DMA,
        pltpu.SemaphoreType.DMA,
        pltpu.SemaphoreType.DMA,
        pltpu.SemaphoreType.DMA,
        pltpu.SemaphoreType.DMA,
    ],
)(_sc_agg_body)


BR = 2048


def _dinv_of(deg_ref):
    deg = deg_ref[0, :, 0] + deg_ref[1, :, 0]
    return jnp.where(deg > 0, lax.rsqrt(deg), 0.0)


def _tc_pre_body(x_ref, w_ref, deg_ref, o_ref):
    dinv = _dinv_of(deg_ref)
    h = jnp.dot(x_ref[...], w_ref[...], preferred_element_type=jnp.float32)
    o_ref[...] = h * dinv[:, None]


def _tc_mid_body(acc_ref, deg_ref, b_ref, w_ref, o_ref):
    dinv = _dinv_of(deg_ref)
    t = (acc_ref[0] + acc_ref[1]) * dinv[:, None] + b_ref[...]
    r = jnp.maximum(t, 0.0)
    o_ref[...] = jnp.dot(r, w_ref[...],
                         preferred_element_type=jnp.float32) * dinv[:, None]


def _tc_post_body(acc_ref, deg_ref, b_ref, o_ref):
    dinv = _dinv_of(deg_ref)
    o_ref[...] = (acc_ref[0] + acc_ref[1]) * dinv[:, None] + b_ref[...]


_acc_spec = pl.BlockSpec((NC, BR, D), lambda i: (0, i, 0))
_deg_spec = _acc_spec
_row_spec = pl.BlockSpec((BR, D), lambda i: (i, 0))
_w_spec = pl.BlockSpec((D, D), lambda i: (0, 0))
_b_spec = pl.BlockSpec((1, D), lambda i: (0, 0))

_tc_pre = pl.pallas_call(
    _tc_pre_body,
    grid=(NP // BR,),
    in_specs=[_row_spec, _w_spec, _deg_spec],
    out_specs=_row_spec,
    out_shape=jax.ShapeDtypeStruct((NP, D), jnp.float32),
)

_tc_mid = pl.pallas_call(
    _tc_mid_body,
    grid=(NP // BR,),
    in_specs=[_acc_spec, _deg_spec, _b_spec, _w_spec],
    out_specs=_row_spec,
    out_shape=jax.ShapeDtypeStruct((NP, D), jnp.float32),
)

_tc_post = pl.pallas_call(
    _tc_post_body,
    grid=(NP // BR,),
    in_specs=[_acc_spec, _deg_spec, _b_spec],
    out_specs=_row_spec,
    out_shape=jax.ShapeDtypeStruct((NP, D), jnp.float32),
)


def kernel(x, edge_index, W1, b1, W2, b2):
    pad = E_PAD - E
    pad_ar = jnp.arange(pad, dtype=jnp.int32)
    src = jnp.concatenate([
        edge_index[0].astype(jnp.int32),
        pad_ar % N,
    ]).reshape(NC, NS, NBLK, 1, IB, CHUNK)
    dst = jnp.concatenate([
        edge_index[1].astype(jnp.int32),
        N + pad_ar % (NP - N),
    ]).reshape(NC, NS, NBLK, 1, IB, CHUNK)
    ed = jnp.concatenate([src, dst], axis=3)

    x_pad = jnp.pad(x, ((0, NP - N), (0, 0)))
    zeros_d = jnp.zeros((NP, D), jnp.float32)
    ones_d = jnp.ones((NP, D), jnp.float32)
    b1r = b1.reshape(1, D)
    b2r = b2.reshape(1, D)

    ones_chunk = jnp.ones((CHUNK, D), jnp.float32)
    deg_parts = _sc_deg(ed, ones_chunk, ones_d, zeros_d)
    h1 = _tc_pre(x_pad, W1, deg_parts)
    acc1 = _sc_agg(h1, ed, zeros_d)
    h2 = _tc_mid(acc1, deg_parts, b1r, W2)
    acc2 = _sc_agg(h2, ed, zeros_d)
    out = _tc_post(acc2, deg_parts, b2r)
    return out[:N]
